# baseline (speedup 1.0000x reference)
import functools

import jax
import jax.numpy as jnp
from jax import lax
from jax.experimental import pallas as pl
from jax.experimental.pallas import tpu as pltpu
from jax.experimental.pallas import tpu_sc as plsc

_N = 10000
_E = 320000
_R = 8
_H = 128

_NC = 2
_NS = 16
_NW = _NC * _NS
_GSZ = 128
_GPT = 80
_EP = _NW * _GPT * _GSZ
_NP = _N + 16
_CHROWS = 80
_NCHUNK = _N // _CHROWS

_BN1 = 1000
_NB1 = _N // _BN1
_BN2 = 1000


def _tc1_body(h_ref, w_ref, q_ref, k_ref, xw_ref, qt_ref, kt_ref):
    xb = jnp.dot(h_ref[...], w_ref[0], preferred_element_type=jnp.float32)
    xw_ref[...] = xb
    qc = jnp.dot(xb, q_ref[...], preferred_element_type=jnp.float32)
    kc = jnp.dot(xb, k_ref[...], preferred_element_type=jnp.float32)
    qt_ref[...] = qc
    kt_ref[...] = kc


@functools.cache
def _tc1_call():
    return pl.pallas_call(
        _tc1_body,
        grid=(_NB1, _R),
        in_specs=[
            pl.BlockSpec((_BN1, _H), lambda nb, r: (nb, 0)),
            pl.BlockSpec((1, _H, _H), lambda nb, r: (r, 0, 0)),
            pl.BlockSpec((_H, 1), lambda nb, r: (0, 0)),
            pl.BlockSpec((_H, 1), lambda nb, r: (0, 0)),
        ],
        out_specs=[
            pl.BlockSpec((_BN1, _H), lambda nb, r: (r * _NB1 + nb, 0)),
            pl.BlockSpec((_BN1, 1), lambda nb, r: (r * _NB1 + nb, 0)),
            pl.BlockSpec((_BN1, 1), lambda nb, r: (r * _NB1 + nb, 0)),
        ],
        out_shape=[
            jax.ShapeDtypeStruct((_R * _N, _H), jnp.float32),
            jax.ShapeDtypeStruct((_R * _N, 1), jnp.float32),
            jax.ShapeDtypeStruct((_R * _N, 1), jnp.float32),
        ],
    )


def _sc_body(xw_hbm, qt_hbm, kt_hbm, idxs_hbm, idxd_hbm, dst_hbm, gb_hbm,
             acc_out, dsum_out,
             acc_sp,
             rows0, rows1, is0, is1, id0, id1, dv0, dv1, qv0, qv1, kv0, kv1,
             s_v, gb_v, d_v, sg0, sg1, ss0, ss1):
    cid = lax.axis_index("c")
    sid = lax.axis_index("s")
    wid = cid * _NS + sid

    pltpu.sync_copy(gb_hbm, gb_v)

    zero16 = jnp.zeros((16,), jnp.float32)
    rows = (rows0, rows1)
    isv = (is0, is1)
    idv = (id0, id1)
    dsv = (dv0, dv1)
    qvv = (qv0, qv1)
    kvv = (kv0, kv1)
    sg = (sg0, sg1)
    ss = (ss0, ss1)

    def _zero_rows(j, c_):
        for c in range(_H // 16):
            rows0[j, pl.ds(c * 16, 16)] = zero16
        return c_

    lax.fori_loop(0, _GSZ, _zero_rows, 0)

    def _zero_d(j, c_):
        d_v[pl.ds(16 * j, 16)] = zero16
        return c_

    lax.fori_loop(0, _NP // 16, _zero_d, 0)

    nk = (_NCHUNK - sid + _NS - 1) // _NS

    def _zero_acc(kk, c_):
        r0 = (sid + kk * _NS) * _CHROWS
        pltpu.sync_copy(rows0.at[pl.ds(0, _CHROWS)],
                        acc_sp.at[pl.ds(r0, _CHROWS)])
        return c_

    lax.fori_loop(0, nk, _zero_acc, 0)
    plsc.subcore_barrier()

    gb = gb_v[pl.ds(0, 16)]
    g_lo = wid * _GPT
    g_end = g_lo + _GPT

    def _start_group(slot, g):
        base = g * _GSZ
        pltpu.sync_copy(idxs_hbm.at[pl.ds(base, _GSZ)], isv[slot])
        pltpu.sync_copy(idxd_hbm.at[pl.ds(base, _GSZ)], idv[slot])
        pltpu.sync_copy(dst_hbm.at[pl.ds(base, _GSZ)], dsv[slot])
        pltpu.async_copy(xw_hbm.at[isv[slot]], rows[slot], sg[slot])
        pltpu.async_copy(qt_hbm.at[idv[slot]], qvv[slot], sg[slot])
        pltpu.async_copy(kt_hbm.at[isv[slot]], kvv[slot], sg[slot])

    def _wait_group(slot):
        pltpu.make_async_copy(xw_hbm.at[isv[slot]], rows[slot],
                              sg[slot]).wait()
        pltpu.make_async_copy(qt_hbm.at[idv[slot]], qvv[slot],
                              sg[slot]).wait()
        pltpu.make_async_copy(kt_hbm.at[isv[slot]], kvv[slot],
                              sg[slot]).wait()

    def _compute(slot):
        for i in range(_GSZ // 16):
            q16 = qvv[slot][pl.ds(16 * i, 16)]
            k16 = kvv[slot][pl.ds(16 * i, 16)]
            a = q16 + k16
            a = jnp.where(a > 0, a, 0.2 * a)
            s = jnp.exp(a - gb)
            s_v[pl.ds(16 * i, 16)] = s
            plsc.addupdate_scatter(d_v, [dsv[slot][pl.ds(16 * i, 16)]], s)

        def _scale(j, cc):
            sj = s_v[pl.ds(j, 16)][0]
            for c in range(_H // 16):
                sl = pl.ds(c * 16, 16)
                rows[slot][j, sl] = rows[slot][j, sl] * sj
            return cc

        lax.fori_loop(0, _GSZ, _scale, 0)

    def _start_scatter(slot):
        pltpu.async_copy(rows[slot], acc_sp.at[dsv[slot]], ss[slot],
                         add=True)

    def _wait_scatter(slot):
        pltpu.make_async_copy(rows[slot], acc_sp.at[dsv[slot]],
                              ss[slot]).wait()

    _start_group(0, g_lo)
    _start_group(1, g_lo + 1)

    def _pair(p, c_):
        g = g_lo + 2 * p
        for slot in (0, 1):
            _wait_group(slot)
            _compute(slot)
            _start_scatter(slot)
            gn = g + slot + 2

            @pl.when(gn < g_end)
            def _():
                _wait_scatter(slot)
                _start_group(slot, gn)
        return c_

    lax.fori_loop(0, _GPT // 2, _pair, 0)
    _wait_scatter(0)
    _wait_scatter(1)
    plsc.subcore_barrier()

    def _drain(kk, c_):
        r0 = (sid + kk * _NS) * _CHROWS
        pltpu.sync_copy(acc_sp.at[pl.ds(r0, _CHROWS)],
                        acc_out.at[cid, pl.ds(r0, _CHROWS)])
        return c_

    lax.fori_loop(0, nk, _drain, 0)
    pltpu.sync_copy(d_v, dsum_out.at[cid, sid])


@functools.cache
def _sc_call():
    mesh = plsc.VectorSubcoreMesh(core_axis_name="c", subcore_axis_name="s",
                                  num_cores=_NC, num_subcores=_NS)
    return pl.kernel(
        _sc_body,
        out_type=[
            jax.ShapeDtypeStruct((_NC, _N, _H), jnp.float32),
            jax.ShapeDtypeStruct((_NC, _NS, _NP), jnp.float32),
        ],
        mesh=mesh,
        compiler_params=pltpu.CompilerParams(needs_layout_passes=False),
        scratch_types=[
            pltpu.VMEM_SHARED((_NP, _H), jnp.float32),
            pltpu.VMEM((_GSZ, _H), jnp.float32),
            pltpu.VMEM((_GSZ, _H), jnp.float32),
            pltpu.VMEM((_GSZ,), jnp.int32),
            pltpu.VMEM((_GSZ,), jnp.int32),
            pltpu.VMEM((_GSZ,), jnp.int32),
            pltpu.VMEM((_GSZ,), jnp.int32),
            pltpu.VMEM((_GSZ,), jnp.int32),
            pltpu.VMEM((_GSZ,), jnp.int32),
            pltpu.VMEM((_GSZ,), jnp.float32),
            pltpu.VMEM((_GSZ,), jnp.float32),
            pltpu.VMEM((_GSZ,), jnp.float32),
            pltpu.VMEM((_GSZ,), jnp.float32),
            pltpu.VMEM((_GSZ + 16,), jnp.float32),
            pltpu.VMEM((128,), jnp.float32),
            pltpu.VMEM((_NP,), jnp.float32),
            pltpu.SemaphoreType.DMA,
            pltpu.SemaphoreType.DMA,
            pltpu.SemaphoreType.DMA,
            pltpu.SemaphoreType.DMA,
        ],
    )


def _tc2_body_plain(acc_ref, d_ref, b_ref, out_ref):
    a = acc_ref[0] + acc_ref[1]
    d = d_ref[...]
    v = jnp.where(d > 0, a / d, 0.0) + b_ref[...]
    out_ref[...] = jnp.maximum(v, 0.0)


def _tc2_body_linear(acc_ref, d_ref, b_ref, lw_ref, lb_ref, out_ref):
    a = acc_ref[0] + acc_ref[1]
    d = d_ref[...]
    v = jnp.where(d > 0, a / d, 0.0) + b_ref[...]
    v = jnp.maximum(v, 0.0)
    out_ref[...] = (jnp.dot(v, lw_ref[...], preferred_element_type=jnp.float32)
                    + lb_ref[...])


@functools.cache
def _tc2_call(out_ch):
    nb2 = _N // _BN2
    common = [
        pl.BlockSpec((_NC, _BN2, _H), lambda nb: (0, nb, 0)),
        pl.BlockSpec((_BN2, 1), lambda nb: (nb, 0)),
        pl.BlockSpec((1, _H), lambda nb: (0, 0)),
    ]
    if out_ch is None:
        return pl.pallas_call(
            _tc2_body_plain,
            grid=(nb2,),
            in_specs=common,
            out_specs=pl.BlockSpec((_BN2, _H), lambda nb: (nb, 0)),
            out_shape=jax.ShapeDtypeStruct((_N, _H), jnp.float32),
        )
    return pl.pallas_call(
        _tc2_body_linear,
        grid=(nb2,),
        in_specs=common + [
            pl.BlockSpec((_H, out_ch), lambda nb: (0, 0)),
            pl.BlockSpec((1, out_ch), lambda nb: (0, 0)),
        ],
        out_specs=pl.BlockSpec((_BN2, out_ch), lambda nb: (nb, 0)),
        out_shape=jax.ShapeDtypeStruct((_N, out_ch), jnp.float32),
    )


def _conv(h, idx_s, idx_d, dst, w, q, k, b, lw=None, lb=None):
    xw, qt, kt = _tc1_call()(h, w, q, k)
    qt = qt.reshape(_R * _N)
    kt = kt.reshape(_R * _N)
    gb = jnp.max(qt) + jnp.max(kt)
    gb128 = jnp.full((128,), gb, jnp.float32)
    acc, dsum = _sc_call()(xw, qt, kt, idx_s, idx_d, dst, gb128)
    d = jnp.sum(dsum, axis=(0, 1))[:_N].reshape(_N, 1)
    b2 = b.reshape(1, _H)
    if lw is None:
        return _tc2_call(None)(acc, d, b2)
    return _tc2_call(lw.shape[1])(acc, d, b2, lw, lb.reshape(1, -1))


def kernel(x, edge_index, edge_type, w1, q1, k1, b1, w_mu, q_mu, k_mu, b_mu,
           w_lv, q_lv, k_lv, b_lv, lw_mu, lb_mu, lw_lv, lb_lv):
    src = edge_index[0]
    dst = edge_index[1]
    t = edge_type
    pad = _EP - _E
    zpad = jnp.zeros((pad,), jnp.int32)
    idx_s = jnp.concatenate([t * _N + src, zpad])
    idx_d = jnp.concatenate([t * _N + dst, zpad])
    dst = jnp.concatenate([dst, jnp.full((pad,), _N, jnp.int32)])
    hidden = _conv(x, idx_s, idx_d, dst, w1, q1, k1, b1)
    mu = _conv(hidden, idx_s, idx_d, dst, w_mu, q_mu, k_mu, b_mu,
               lw=lw_mu, lb=lb_mu)
    logstd = _conv(hidden, idx_s, idx_d, dst, w_lv, q_lv, k_lv, b_lv,
                   lw=lw_lv * 0.5, lb=lb_lv * 0.5)
    return (mu, logstd)


# speedup vs baseline: 1.4311x; 1.4311x over previous
import functools

import jax
import jax.numpy as jnp
from jax import lax
from jax.experimental import pallas as pl
from jax.experimental.pallas import tpu as pltpu
from jax.experimental.pallas import tpu_sc as plsc

_N = 10000
_E = 320000
_R = 8
_H = 128

_NC = 2
_NS = 16
_NW = _NC * _NS
_GSZ = 128
_NG = _E // _GSZ
_CHROWS = 80
_NCHUNK = _N // _CHROWS

_BN1 = 1000
_NB1 = _N // _BN1
_BN2 = 1000


def _tc1_body(h_ref, w_ref, q_ref, k_ref, xw_ref, qt_ref, kt_ref):
    xb = jnp.dot(h_ref[...], w_ref[0], preferred_element_type=jnp.float32)
    xw_ref[...] = xb
    qc = jnp.dot(xb, q_ref[...], preferred_element_type=jnp.float32)
    kc = jnp.dot(xb, k_ref[...], preferred_element_type=jnp.float32)
    qt_ref[...] = qc
    kt_ref[...] = kc


@functools.cache
def _tc1_call():
    return pl.pallas_call(
        _tc1_body,
        grid=(_NB1, _R),
        in_specs=[
            pl.BlockSpec((_BN1, _H), lambda nb, r: (nb, 0)),
            pl.BlockSpec((1, _H, _H), lambda nb, r: (r, 0, 0)),
            pl.BlockSpec((_H, 1), lambda nb, r: (0, 0)),
            pl.BlockSpec((_H, 1), lambda nb, r: (0, 0)),
        ],
        out_specs=[
            pl.BlockSpec((_BN1, _H), lambda nb, r: (r * _NB1 + nb, 0)),
            pl.BlockSpec((_BN1, 1), lambda nb, r: (r * _NB1 + nb, 0)),
            pl.BlockSpec((_BN1, 1), lambda nb, r: (r * _NB1 + nb, 0)),
        ],
        out_shape=[
            jax.ShapeDtypeStruct((_R * _N, _H), jnp.float32),
            jax.ShapeDtypeStruct((_R * _N, 1), jnp.float32),
            jax.ShapeDtypeStruct((_R * _N, 1), jnp.float32),
        ],
    )


def _sc_body(xw_hbm, qt_hbm, kt_hbm, idxs_hbm, idxd_hbm, dst_hbm, gb_hbm,
             acc_out, dsum_out,
             acc_sp,
             rows_v, qrow_v, krow_v, idxs_v, idxd_v, dstv_v, s_v, gb_v, d_v,
             sem_e, sem_g):
    cid = lax.axis_index("c")
    sid = lax.axis_index("s")
    wid = cid * _NS + sid

    pltpu.sync_copy(gb_hbm, gb_v)

    zero16 = jnp.zeros((16,), jnp.float32)

    def _zero_rows(j, c_):
        for c in range(_H // 16):
            rows_v[j, pl.ds(c * 16, 16)] = zero16
        return c_

    lax.fori_loop(0, _GSZ, _zero_rows, 0)

    def _zero_d(j, c_):
        d_v[pl.ds(16 * j, 16)] = zero16
        return c_

    lax.fori_loop(0, _N // 16, _zero_d, 0)

    nk = (_NCHUNK - sid + _NS - 1) // _NS

    def _zero_acc(kk, c_):
        r0 = (sid + kk * _NS) * _CHROWS
        pltpu.sync_copy(rows_v.at[pl.ds(0, _CHROWS)],
                        acc_sp.at[pl.ds(r0, _CHROWS)])
        return c_

    lax.fori_loop(0, nk, _zero_acc, 0)
    plsc.subcore_barrier()

    gb = gb_v[pl.ds(0, 16)]

    g_lo = (wid * _NG) // _NW
    g_hi = ((wid + 1) * _NG) // _NW

    def _group(g, c_):
        base = g * _GSZ
        pltpu.async_copy(idxs_hbm.at[pl.ds(base, _GSZ)], idxs_v, sem_e)
        pltpu.async_copy(idxd_hbm.at[pl.ds(base, _GSZ)], idxd_v, sem_e)
        pltpu.async_copy(dst_hbm.at[pl.ds(base, _GSZ)], dstv_v, sem_e)
        pltpu.make_async_copy(idxs_hbm.at[pl.ds(base, _GSZ)], idxs_v,
                              sem_e).wait()
        pltpu.make_async_copy(idxd_hbm.at[pl.ds(base, _GSZ)], idxd_v,
                              sem_e).wait()
        pltpu.make_async_copy(dst_hbm.at[pl.ds(base, _GSZ)], dstv_v,
                              sem_e).wait()
        pltpu.async_copy(xw_hbm.at[idxs_v], rows_v, sem_g)
        pltpu.async_copy(qt_hbm.at[idxd_v], qrow_v, sem_g)
        pltpu.async_copy(kt_hbm.at[idxs_v], krow_v, sem_g)
        pltpu.make_async_copy(xw_hbm.at[idxs_v], rows_v, sem_g).wait()
        pltpu.make_async_copy(qt_hbm.at[idxd_v], qrow_v, sem_g).wait()
        pltpu.make_async_copy(kt_hbm.at[idxs_v], krow_v, sem_g).wait()
        for i in range(_GSZ // 16):
            q16 = qrow_v[pl.ds(16 * i, 16)]
            k16 = krow_v[pl.ds(16 * i, 16)]
            a = q16 + k16
            a = jnp.where(a > 0, a, 0.2 * a)
            s = jnp.exp(a - gb)
            s_v[pl.ds(16 * i, 16)] = s
            plsc.addupdate_scatter(d_v, [dstv_v[pl.ds(16 * i, 16)]], s)

        def _scale(j, cc):
            sj = s_v[pl.ds(j, 16)][0]
            for c in range(_H // 16):
                sl = pl.ds(c * 16, 16)
                rows_v[j, sl] = rows_v[j, sl] * sj
            return cc

        lax.fori_loop(0, _GSZ, _scale, 0)
        pltpu.sync_copy(rows_v, acc_sp.at[dstv_v], add=True)
        return c_

    lax.fori_loop(g_lo, g_hi, _group, 0)
    plsc.subcore_barrier()

    def _drain(kk, c_):
        r0 = (sid + kk * _NS) * _CHROWS
        pltpu.sync_copy(acc_sp.at[pl.ds(r0, _CHROWS)],
                        acc_out.at[cid, pl.ds(r0, _CHROWS)])
        return c_

    lax.fori_loop(0, nk, _drain, 0)
    pltpu.sync_copy(d_v, dsum_out.at[cid, sid])


@functools.cache
def _sc_call():
    mesh = plsc.VectorSubcoreMesh(core_axis_name="c", subcore_axis_name="s",
                                  num_cores=_NC, num_subcores=_NS)
    return pl.kernel(
        _sc_body,
        out_type=[
            jax.ShapeDtypeStruct((_NC, _N, _H), jnp.float32),
            jax.ShapeDtypeStruct((_NC, _NS, _N), jnp.float32),
        ],
        mesh=mesh,
        compiler_params=pltpu.CompilerParams(needs_layout_passes=False),
        scratch_types=[
            pltpu.VMEM_SHARED((_N, _H), jnp.float32),
            pltpu.VMEM((_GSZ, _H), jnp.float32),
            pltpu.VMEM((_GSZ,), jnp.float32),
            pltpu.VMEM((_GSZ,), jnp.float32),
            pltpu.VMEM((_GSZ,), jnp.int32),
            pltpu.VMEM((_GSZ,), jnp.int32),
            pltpu.VMEM((_GSZ,), jnp.int32),
            pltpu.VMEM((_GSZ + 16,), jnp.float32),
            pltpu.VMEM((128,), jnp.float32),
            pltpu.VMEM((_N,), jnp.float32),
            pltpu.SemaphoreType.DMA,
            pltpu.SemaphoreType.DMA,
        ],
    )


def _tc2_body_plain(acc_ref, d_ref, b_ref, out_ref):
    a = acc_ref[0] + acc_ref[1]
    d = d_ref[...]
    v = jnp.where(d > 0, a / d, 0.0) + b_ref[...]
    out_ref[...] = jnp.maximum(v, 0.0)


def _tc2_body_linear(acc_ref, d_ref, b_ref, lw_ref, lb_ref, out_ref):
    a = acc_ref[0] + acc_ref[1]
    d = d_ref[...]
    v = jnp.where(d > 0, a / d, 0.0) + b_ref[...]
    v = jnp.maximum(v, 0.0)
    out_ref[...] = (jnp.dot(v, lw_ref[...], preferred_element_type=jnp.float32)
                    + lb_ref[...])


@functools.cache
def _tc2_call(out_ch):
    nb2 = _N // _BN2
    common = [
        pl.BlockSpec((_NC, _BN2, _H), lambda nb: (0, nb, 0)),
        pl.BlockSpec((_BN2, 1), lambda nb: (nb, 0)),
        pl.BlockSpec((1, _H), lambda nb: (0, 0)),
    ]
    if out_ch is None:
        return pl.pallas_call(
            _tc2_body_plain,
            grid=(nb2,),
            in_specs=common,
            out_specs=pl.BlockSpec((_BN2, _H), lambda nb: (nb, 0)),
            out_shape=jax.ShapeDtypeStruct((_N, _H), jnp.float32),
        )
    return pl.pallas_call(
        _tc2_body_linear,
        grid=(nb2,),
        in_specs=common + [
            pl.BlockSpec((_H, out_ch), lambda nb: (0, 0)),
            pl.BlockSpec((1, out_ch), lambda nb: (0, 0)),
        ],
        out_specs=pl.BlockSpec((_BN2, out_ch), lambda nb: (nb, 0)),
        out_shape=jax.ShapeDtypeStruct((_N, out_ch), jnp.float32),
    )


def _conv(h, idx_s, idx_d, dst, w, q, k, b, lw=None, lb=None):
    xw, qt, kt = _tc1_call()(h, w, q, k)
    qt = qt.reshape(_R * _N)
    kt = kt.reshape(_R * _N)
    gb = jnp.max(qt) + jnp.max(kt)
    gb128 = jnp.full((128,), gb, jnp.float32)
    acc, dsum = _sc_call()(xw, qt, kt, idx_s, idx_d, dst, gb128)
    d = jnp.sum(dsum, axis=(0, 1)).reshape(_N, 1)
    b2 = b.reshape(1, _H)
    if lw is None:
        return _tc2_call(None)(acc, d, b2)
    return _tc2_call(lw.shape[1])(acc, d, b2, lw, lb.reshape(1, -1))


def kernel(x, edge_index, edge_type, w1, q1, k1, b1, w_mu, q_mu, k_mu, b_mu,
           w_lv, q_lv, k_lv, b_lv, lw_mu, lb_mu, lw_lv, lb_lv):
    src = edge_index[0]
    dst = edge_index[1]
    t = edge_type
    idx_s = t * _N + src
    idx_d = t * _N + dst
    hidden = _conv(x, idx_s, idx_d, dst, w1, q1, k1, b1)
    mu = _conv(hidden, idx_s, idx_d, dst, w_mu, q_mu, k_mu, b_mu,
               lw=lw_mu, lb=lb_mu)
    logstd = _conv(hidden, idx_s, idx_d, dst, w_lv, q_lv, k_lv, b_lv,
                   lw=lw_lv * 0.5, lb=lb_lv * 0.5)
    return (mu, logstd)
